# baseline (speedup 1.0000x reference)
import jax
import jax.numpy as jnp
from jax.experimental import pallas as pl

N = 16384
D = 2048
R = 512
NB = N // R


def _sumsq_body(hs_ref, out_ref):
    x = hs_ref[...]
    out_ref[0, 0, :] = jnp.sum(x * x, axis=-1)


def _scale_body(sumsq_full_ref, sumsq_blk_ref, ids_blk_ref, hs_ref, out_ref):
    norms = jnp.sqrt(sumsq_full_ref[...])
    mean = jnp.sum(norms) / N
    dev = norms - mean
    std = jnp.sqrt(jnp.sum(dev * dev) / (N - 1))
    thr = mean + 2.0 * std

    my_norm = jnp.sqrt(sumsq_blk_ref[0, 0, :])
    my_ids = ids_blk_ref[0, 0, :]
    leak = (my_ids <= 60) & (my_norm > thr)
    factor = jnp.where(leak & (my_ids <= 40), jnp.float32(0.1),
                       jnp.where(leak, jnp.float32(0.5), jnp.float32(1.0)))
    out_ref[...] = hs_ref[...] * factor[:, None]


def kernel(hidden_states, namespace_ids):
    B, S, Dh = hidden_states.shape
    hs = hidden_states.reshape(N, D)
    ids = namespace_ids.reshape(NB, 1, R)

    sumsq = pl.pallas_call(
        _sumsq_body,
        grid=(NB,),
        in_specs=[pl.BlockSpec((R, D), lambda i: (i, 0))],
        out_specs=pl.BlockSpec((1, 1, R), lambda i: (i, 0, 0)),
        out_shape=jax.ShapeDtypeStruct((NB, 1, R), jnp.float32),
    )(hs)

    out = pl.pallas_call(
        _scale_body,
        grid=(NB,),
        in_specs=[
            pl.BlockSpec((NB, 1, R), lambda i: (0, 0, 0)),
            pl.BlockSpec((1, 1, R), lambda i: (i, 0, 0)),
            pl.BlockSpec((1, 1, R), lambda i: (i, 0, 0)),
            pl.BlockSpec((R, D), lambda i: (i, 0)),
        ],
        out_specs=pl.BlockSpec((R, D), lambda i: (i, 0)),
        out_shape=jax.ShapeDtypeStruct((N, D), jnp.float32),
    )(sumsq, sumsq, ids, hs)

    return out.reshape(B, S, Dh)
